# baseline (speedup 1.0000x reference)
import jax
import jax.numpy as jnp
from jax import lax
from jax.experimental import pallas as pl
from jax.experimental.pallas import tpu as pltpu
from jax.experimental.pallas import tpu_sc as plsc

N = 10000
E = 320000
D = 128
DE = 16
H = 128
EMB = 128
G = 64

NC = 2
NS = 16
NW = NC * NS

CK = 128
CKG = 512
NCHUNKG = E // CKG
RFULLG = NCHUNKG // NW
RREMG = NCHUNKG - RFULLG * NW
NCHUNK = E // CK
RFULL = NCHUNK // NW
RREM = NCHUNK - RFULL * NW

NPW = 320
NP = NPW * NW
GP = 72

NDP = 10240
NTILE = 624
NREST = N - NTILE * NS
_HOPS = [(k, CK) for k in range(NTILE // CK)]
if NTILE % CK:
    _HOPS.append((NTILE // CK, NTILE % CK))

NBLK = 10
EBLK = 64


def _mesh():
    return plsc.VectorSubcoreMesh(core_axis_name="c", subcore_axis_name="s")


CKB = 1280
NCHUNKB = E // CKB
RFULLB = NCHUNKB // NW
RREMB = NCHUNKB - RFULLB * NW


def _sc_phaseB_body(ei_hbm, elog_hbm, s_hbm, d_hbm, exv_hbm, denp_hbm,
                    s_v, d_v, den_v, sdb, elb, exb, dstage, densum, den_sh):
    c = lax.axis_index("c")
    s = lax.axis_index("s")
    wid = s * NC + c

    pltpu.sync_copy(s_hbm, s_v)
    pltpu.sync_copy(d_hbm, d_v)

    zero16 = jnp.zeros((16,), jnp.float32)

    def _zero_den(g, carry):
        den_v[pl.ds(g * 16, 16)] = zero16
        return carry
    lax.fori_loop(0, NDP // 16, _zero_den, 0)

    def _chunk(chunk_id):
        base = chunk_id * CKB
        pltpu.sync_copy(ei_hbm.at[:, pl.ds(base, CKB)], sdb)
        pltpu.sync_copy(elog_hbm.at[pl.ds(base, CKB)], elb)

        def _grp(g, carry):
            off = g * 16
            src16 = sdb[0, pl.ds(off, 16)]
            dst16 = sdb[1, pl.ds(off, 16)]
            l16 = (plsc.load_gather(s_v, [src16])
                   + plsc.load_gather(d_v, [dst16])
                   + elb[pl.ds(off, 16)])
            l16 = jnp.where(l16 >= 0.0, l16, l16 * 0.2)
            ex16 = jnp.exp(l16)
            exb[pl.ds(off, 16)] = ex16
            plsc.addupdate_scatter(den_v, [dst16], ex16)
            return carry
        lax.fori_loop(0, CKB // 16, _grp, 0)
        pltpu.sync_copy(exb, exv_hbm.at[pl.ds(base, CKB)])

    def _round(r, carry):
        _chunk(r * NW + wid)
        return carry
    lax.fori_loop(0, RFULLB, _round, 0)

    @pl.when(wid < RREMB)
    def _tail():
        _chunk(RFULLB * NW + wid)

    pltpu.sync_copy(den_v, den_sh.at[s])
    plsc.subcore_barrier()
    r0 = s * 640
    pltpu.sync_copy(den_sh.at[:, pl.ds(r0, 640)], dstage)

    def _redgrp(g, carry):
        acc = dstage[0, pl.ds(g * 16, 16)]
        for tt in range(1, NS):
            acc = acc + dstage[tt, pl.ds(g * 16, 16)]
        densum[pl.ds(g * 16, 16)] = acc
        return carry
    lax.fori_loop(0, 40, _redgrp, 0)
    pltpu.sync_copy(densum, denp_hbm.at[c, 0, pl.ds(r0, 640)])


def _sc_phaseB(ei, elog, s, d):
    k = pl.kernel(
        _sc_phaseB_body,
        out_type=(
            jax.ShapeDtypeStruct((E,), jnp.float32),
            jax.ShapeDtypeStruct((NC, 1, NDP), jnp.float32),
        ),
        mesh=_mesh(),
        compiler_params=pltpu.CompilerParams(needs_layout_passes=False),
        scratch_types=[
            pltpu.VMEM((N,), jnp.float32),
            pltpu.VMEM((N,), jnp.float32),
            pltpu.VMEM((NDP,), jnp.float32),
            pltpu.VMEM((2, CKB), jnp.int32),
            pltpu.VMEM((CKB,), jnp.float32),
            pltpu.VMEM((CKB,), jnp.float32),
            pltpu.VMEM((NS, 640), jnp.float32),
            pltpu.VMEM((640,), jnp.float32),
            pltpu.VMEM_SHARED((NS, NDP), jnp.float32),
        ],
    )
    return k(ei, elog, s, d)


_NCH = NCHUNK
_RF = _NCH // NW
_RREMC = _NCH - _RF * NW


def _sc_phaseC_body(ei_hbm, exv_hbm, h_hbm, outp_hbm,
                    sdb2, dsc2, exb2, rows2, out_sh,
                    sem_g0, sem_g1, sem_s0, sem_s1, sem_i0, sem_i1):
    c = lax.axis_index("c")
    s = lax.axis_index("s")
    wid = s * NC + c
    sem_g = (sem_g0, sem_g1)
    sem_s = (sem_s0, sem_s1)
    sem_i = (sem_i0, sem_i1)

    zero16 = jnp.zeros((16,), jnp.float32)

    def _zero_rows(r, carry):
        for j in range(8):
            rows2[0, r, pl.ds(j * 16, 16)] = zero16
        return carry
    lax.fori_loop(0, CK, _zero_rows, 0)

    for k, nr in _HOPS:
        pltpu.sync_copy(rows2.at[0, pl.ds(0, nr)],
                        out_sh.at[pl.ds(s * NTILE + k * CK, nr)])

    @pl.when(s == 0)
    def _zero_rest():
        pltpu.sync_copy(rows2.at[0, pl.ds(0, NREST)],
                        out_sh.at[pl.ds(NS * NTILE, NREST)])
    plsc.subcore_barrier()

    def _cid(k):
        return k * NW + wid

    def _fire_stages(k, b):
        base = _cid(k) * CK
        pltpu.async_copy(ei_hbm.at[:, pl.ds(base, CK)], sdb2.at[b], sem_i[b])
        pltpu.async_copy(exv_hbm.at[pl.ds(base, CK)],
                         exb2.at[b, pl.ds(0, CK)], sem_i[b])

    def _drain_stages(b):
        pltpu.make_async_copy(ei_hbm.at[:, pl.ds(0, CK)], sdb2.at[b],
                              sem_i[b]).wait()
        pltpu.make_async_copy(exv_hbm.at[pl.ds(0, CK)],
                              exb2.at[b, pl.ds(0, CK)], sem_i[b]).wait()

    def _fire_gather(k, b):
        pltpu.async_copy(h_hbm.at[sdb2.at[b, 0]], rows2.at[b], sem_g[b])

    def _drain_gather(b):
        pltpu.make_async_copy(h_hbm.at[pl.ds(0, CK)], rows2.at[b],
                              sem_g[b]).wait()

    def _fire_scatter(b):
        pltpu.async_copy(rows2.at[b], out_sh.at[dsc2.at[b, 0]], sem_s[b],
                         add=True)

    def _drain_scatter(b):
        pltpu.make_async_copy(h_hbm.at[pl.ds(0, CK)], rows2.at[b],
                              sem_s[b]).wait()

    def _scale(b):
        def _one(e, carry):
            a16 = jnp.broadcast_to(exb2[b, pl.ds(e, 16)][0], (16,))
            for j in range(8):
                v = rows2[b, e, pl.ds(j * 16, 16)]
                rows2[b, e, pl.ds(j * 16, 16)] = v * a16
            return carry
        lax.fori_loop(0, CK, _one, 0, unroll=4)

    base0 = _cid(0) * CK
    pltpu.sync_copy(ei_hbm.at[:, pl.ds(base0, CK)], sdb2.at[0])
    pltpu.sync_copy(exv_hbm.at[pl.ds(base0, CK)], exb2.at[0, pl.ds(0, CK)])
    _fire_gather(0, 0)
    _fire_stages(1, 1)

    def _pair(r, carry):
        for b in (0, 1):
            k = r * 2 + b
            _drain_gather(b)
            for g in range(CK // 16):
                dsc2[b, 0, pl.ds(g * 16, 16)] = sdb2[b, 1, pl.ds(g * 16, 16)]
            _scale(b)
            _fire_scatter(b)

            @pl.when(k + 2 < _RF)
            def _pf_stages():
                _fire_stages(k + 2, b)

            @pl.when(k + 1 < _RF)
            def _pf_gather():
                @pl.when(k >= 1)
                def _w_sct():
                    _drain_scatter(1 - b)
                _drain_stages(1 - b)
                _fire_gather(k + 1, 1 - b)
        return carry
    lax.fori_loop(0, _RF // 2, _pair, 0)

    _drain_scatter(0)
    _drain_scatter(1)

    @pl.when(wid < _RREMC)
    def _tail():
        base = (_RF * NW + wid) * CK
        pltpu.sync_copy(ei_hbm.at[:, pl.ds(base, CK)], sdb2.at[0])
        pltpu.sync_copy(exv_hbm.at[pl.ds(base, CK)], exb2.at[0, pl.ds(0, CK)])
        pltpu.sync_copy(h_hbm.at[sdb2.at[0, 0]], rows2.at[0])
        for g in range(CK // 16):
            dsc2[0, 0, pl.ds(g * 16, 16)] = sdb2[0, 1, pl.ds(g * 16, 16)]
        _scale(0)
        pltpu.sync_copy(rows2.at[0], out_sh.at[dsc2.at[0, 0]], add=True)

    plsc.subcore_barrier()
    for k, nr in _HOPS:
        r0 = s * NTILE + k * CK
        pltpu.sync_copy(out_sh.at[pl.ds(r0, nr)], rows2.at[0, pl.ds(0, nr)])
        pltpu.sync_copy(rows2.at[0, pl.ds(0, nr)],
                        outp_hbm.at[c, pl.ds(r0, nr)])

    @pl.when(s == 0)
    def _drain_rest():
        pltpu.sync_copy(out_sh.at[pl.ds(NS * NTILE, NREST)],
                        rows2.at[0, pl.ds(0, NREST)])
        pltpu.sync_copy(rows2.at[0, pl.ds(0, NREST)],
                        outp_hbm.at[c, pl.ds(NS * NTILE, NREST)])


def _sc_phaseC(ei, exv, h):
    k = pl.kernel(
        _sc_phaseC_body,
        out_type=jax.ShapeDtypeStruct((NC, N, H), jnp.float32),
        mesh=_mesh(),
        compiler_params=pltpu.CompilerParams(needs_layout_passes=False),
        scratch_types=[
            pltpu.VMEM((2, 2, CK), jnp.int32),
            pltpu.VMEM((2, 1, CK), jnp.int32),
            pltpu.VMEM((2, CK + 16), jnp.float32),
            pltpu.VMEM((2, CK, H), jnp.float32),
            pltpu.VMEM_SHARED((N, H), jnp.float32),
            pltpu.SemaphoreType.DMA,
            pltpu.SemaphoreType.DMA,
            pltpu.SemaphoreType.DMA,
            pltpu.SemaphoreType.DMA,
            pltpu.SemaphoreType.DMA,
            pltpu.SemaphoreType.DMA,
        ],
    )
    return k(ei, exv, h)


def _sc_layer(ei, elog, h, s, d):
    exv, denp = _sc_phaseB(ei, elog, s, d)
    outp = _sc_phaseC(ei, exv, h)
    return outp, denp


def _sc_pool_body(emb_hbm, batch_hbm, psum_hbm, pmax_hbm, pcnt_hbm,
                  ebuf, bbuf, tsum, tmax, tcnt):
    c = lax.axis_index("c")
    s = lax.axis_index("s")
    wid = s * NC + c

    base = jnp.minimum(wid * NPW, N - NPW)
    skip = wid * NPW - base
    pltpu.sync_copy(emb_hbm.at[pl.ds(base, NPW)], ebuf)
    pltpu.sync_copy(batch_hbm.at[pl.ds(base, NPW)], bbuf.at[pl.ds(0, NPW)])

    zero16 = jnp.zeros((16,), jnp.float32)
    neg16 = jnp.full((16,), -3.0e38, jnp.float32)

    def _init(g, carry):
        for j in range(8):
            tsum[g, pl.ds(j * 16, 16)] = zero16
            tmax[g, pl.ds(j * 16, 16)] = neg16
        return carry
    lax.fori_loop(0, GP, _init, 0)

    def _zero_cnt(g, carry):
        tcnt[pl.ds(g * 16, 16)] = zero16
        return carry
    lax.fori_loop(0, 6, _zero_cnt, 0)

    def _node(i, carry):
        lane0 = (lax.iota(jnp.int32, 16) == 0).astype(jnp.float32)
        b = bbuf[pl.ds(i, 16)][0]
        for j in range(8):
            row = ebuf[i, pl.ds(j * 16, 16)]
            ts = tsum[b, pl.ds(j * 16, 16)]
            tsum[b, pl.ds(j * 16, 16)] = ts + row
            tm = tmax[b, pl.ds(j * 16, 16)]
            tmax[b, pl.ds(j * 16, 16)] = jnp.maximum(tm, row)
        cn = tcnt[pl.ds(b, 16)]
        tcnt[pl.ds(b, 16)] = cn + lane0
        return carry
    lax.fori_loop(skip, NPW, _node, 0)

    pltpu.sync_copy(tsum, psum_hbm.at[wid])
    pltpu.sync_copy(tmax, pmax_hbm.at[wid])
    pltpu.sync_copy(tcnt.at[pl.ds(0, GP)], pcnt_hbm.at[wid, 0])


def _sc_pool(emb_p, batch_p):
    k = pl.kernel(
        _sc_pool_body,
        out_type=(
            jax.ShapeDtypeStruct((NW, GP, EMB), jnp.float32),
            jax.ShapeDtypeStruct((NW, GP, EMB), jnp.float32),
            jax.ShapeDtypeStruct((NW, 1, GP), jnp.float32),
        ),
        mesh=_mesh(),
        compiler_params=pltpu.CompilerParams(needs_layout_passes=False),
        scratch_types=[
            pltpu.VMEM((NPW, EMB), jnp.float32),
            pltpu.VMEM((NPW + 16,), jnp.int32),
            pltpu.VMEM((GP, EMB), jnp.float32),
            pltpu.VMEM((GP, EMB), jnp.float32),
            pltpu.VMEM((96,), jnp.float32),
        ],
    )
    return k(emb_p, batch_p)


def _sc_gather2_body(src_hbm, dst_hbm, u1_hbm, u2_hbm, g_hbm,
                     srcb, dstb, rows, sem):
    c = lax.axis_index("c")
    s = lax.axis_index("s")
    wid = s * NC + c

    def _chunk(chunk_id):
        base = chunk_id * CKG
        pltpu.sync_copy(src_hbm.at[pl.ds(base, CKG)], srcb.at[0])
        pltpu.sync_copy(dst_hbm.at[pl.ds(base, CKG)], dstb.at[0])
        pltpu.sync_copy(u1_hbm.at[srcb.at[0]], rows)
        pltpu.async_copy(u2_hbm.at[dstb.at[0]], rows, sem, add=True).wait()
        pltpu.sync_copy(rows, g_hbm.at[pl.ds(base, CKG)])

    def _round(r, carry):
        _chunk(r * NW + wid)
        return carry
    lax.fori_loop(0, RFULLG, _round, 0)

    @pl.when(wid < RREMG)
    def _tail():
        _chunk(RFULLG * NW + wid)


def _sc_gather2(src, dst, u1p, u2):
    k = pl.kernel(
        _sc_gather2_body,
        out_type=jax.ShapeDtypeStruct((E, H), jnp.float32),
        mesh=_mesh(),
        compiler_params=pltpu.CompilerParams(needs_layout_passes=False),
        scratch_types=[
            pltpu.VMEM((1, CKG), jnp.int32),
            pltpu.VMEM((1, CKG), jnp.int32),
            pltpu.VMEM((CKG, H), jnp.float32),
            pltpu.SemaphoreType.DMA,
        ],
    )
    return k(src, dst, u1p, u2)


def _proj_body(act_ref, w_ref, a2_ref, h_ref, sd_ref):
    h = jnp.dot(act_ref[...], w_ref[...], preferred_element_type=jnp.float32)
    h_ref[...] = h
    sd_ref[...] = jnp.dot(h, a2_ref[...], preferred_element_type=jnp.float32)


def _proj(act, w, a_s, a_d):
    a2 = jnp.stack([a_s, a_d], axis=1)
    rb = N // NBLK
    h, sd = pl.pallas_call(
        _proj_body,
        grid=(NBLK,),
        in_specs=[
            pl.BlockSpec((rb, D), lambda i: (i, 0)),
            pl.BlockSpec((D, H), lambda i: (0, 0)),
            pl.BlockSpec((H, 2), lambda i: (0, 0)),
        ],
        out_specs=[
            pl.BlockSpec((rb, H), lambda i: (i, 0)),
            pl.BlockSpec((rb, 2), lambda i: (i, 0)),
        ],
        out_shape=[
            jax.ShapeDtypeStruct((N, H), jnp.float32),
            jax.ShapeDtypeStruct((N, 2), jnp.float32),
        ],
    )(act, w, a2)
    return h, sd[:, 0], sd[:, 1]


def _combine_proj_body(p_ref, dent_ref, b_ref, w_ref, a2_ref, h_ref, sd_ref):
    dr = 1.0 / (jnp.sum(dent_ref[...], axis=1) + 1e-16)
    o = (p_ref[0] + p_ref[1]) * dr[:, None] + b_ref[...][None, :]
    act = jnp.maximum(o, 0.0)
    h = jnp.dot(act, w_ref[...], preferred_element_type=jnp.float32)
    h_ref[...] = h
    sd_ref[...] = jnp.dot(h, a2_ref[...], preferred_element_type=jnp.float32)


def _combine_proj(outp, dent, b, w, a_s, a_d):
    a2 = jnp.stack([a_s, a_d], axis=1)
    rb = N // NBLK
    h, sd = pl.pallas_call(
        _combine_proj_body,
        grid=(NBLK,),
        in_specs=[
            pl.BlockSpec((NC, rb, H), lambda i: (0, i, 0)),
            pl.BlockSpec((rb, NC), lambda i: (i, 0)),
            pl.BlockSpec((H,), lambda i: (0,)),
            pl.BlockSpec((D, H), lambda i: (0, 0)),
            pl.BlockSpec((H, 2), lambda i: (0, 0)),
        ],
        out_specs=[
            pl.BlockSpec((rb, H), lambda i: (i, 0)),
            pl.BlockSpec((rb, 2), lambda i: (i, 0)),
        ],
        out_shape=[
            jax.ShapeDtypeStruct((N, H), jnp.float32),
            jax.ShapeDtypeStruct((N, 2), jnp.float32),
        ],
    )(outp, dent, b, w, a2)
    return h, sd[:, 0], sd[:, 1]


def _combine2_body(p_ref, dent_ref, b_ref, aw_ref, emb_ref, u_ref):
    dr = 1.0 / (jnp.sum(dent_ref[...], axis=1) + 1e-16)
    emb = (p_ref[0] + p_ref[1]) * dr[:, None] + b_ref[...][None, :]
    emb_ref[...] = emb
    u_ref[...] = jnp.dot(emb, aw_ref[...], preferred_element_type=jnp.float32)


def _combine2(outp, dent, b, a1w, a2w):
    aw = jnp.concatenate([a1w, a2w], axis=1)
    rb = N // NBLK
    emb, u = pl.pallas_call(
        _combine2_body,
        grid=(NBLK,),
        in_specs=[
            pl.BlockSpec((NC, rb, H), lambda i: (0, i, 0)),
            pl.BlockSpec((rb, NC), lambda i: (i, 0)),
            pl.BlockSpec((H,), lambda i: (0,)),
            pl.BlockSpec((H, 2 * H), lambda i: (0, 0)),
        ],
        out_specs=[
            pl.BlockSpec((rb, H), lambda i: (i, 0)),
            pl.BlockSpec((rb, 2 * H), lambda i: (i, 0)),
        ],
        out_shape=[
            jax.ShapeDtypeStruct((N, H), jnp.float32),
            jax.ShapeDtypeStruct((N, 2 * H), jnp.float32),
        ],
    )(outp, dent, b, aw)
    return emb, u[:, :H], u[:, H:]


def _ctx_fold_body(u1_ref, b2d_ref, ps_ref, pm_ref, pc_ref, ac_ref,
                   m1b_ref, o_ref):
    cnt = jnp.sum(pc_ref[...], axis=(0, 1))[:G]
    gsum = jnp.sum(ps_ref[...], axis=0)[:G]
    gmax = jnp.max(pm_ref[...], axis=0)[:G]
    gmean = gsum / jnp.clip(cnt, 1.0)[:, None]
    gmax = jnp.where(cnt[:, None] > 0.5, gmax, 0.0)
    ctx = jnp.concatenate([gmean, gmax], axis=1)
    cb = jnp.dot(ctx, ac_ref[...], preferred_element_type=jnp.float32) \
        + m1b_ref[...][None, :]
    oh = (b2d_ref[...] == lax.broadcasted_iota(jnp.int32, (1, G), 1))
    o_ref[...] = u1_ref[...] + jnp.dot(
        oh.astype(jnp.float32), cb, preferred_element_type=jnp.float32)


def _ctx_fold(u1, batch, psum, pmax, pcnt, ac_w, m1b):
    rb = N // NBLK
    return pl.pallas_call(
        _ctx_fold_body,
        grid=(NBLK,),
        in_specs=[
            pl.BlockSpec((rb, H), lambda i: (i, 0)),
            pl.BlockSpec((rb, 1), lambda i: (i, 0)),
            pl.BlockSpec((NW, GP, EMB), lambda i: (0, 0, 0)),
            pl.BlockSpec((NW, GP, EMB), lambda i: (0, 0, 0)),
            pl.BlockSpec((NW, 1, GP), lambda i: (0, 0, 0)),
            pl.BlockSpec((2 * EMB, H), lambda i: (0, 0)),
            pl.BlockSpec((H,), lambda i: (0,)),
        ],
        out_specs=pl.BlockSpec((rb, H), lambda i: (i, 0)),
        out_shape=jax.ShapeDtypeStruct((N, H), jnp.float32),
    )(u1, batch[:, None], psum, pmax, pcnt, ac_w, m1b)


def _elog_body(ea_ref, b_ref, o0_ref, o1_ref, o2_ref):
    o = jnp.dot(ea_ref[...], b_ref[...], preferred_element_type=jnp.float32)
    o0_ref[...] = o[:, 0:8]
    o1_ref[...] = o[:, 8:16]
    o2_ref[...] = o[:, 16:24]


def _elogs(edge_attr, vs):
    ear = edge_attr.reshape(E // 8, 8 * DE)
    b = jnp.zeros((8 * DE, 24), jnp.float32)
    for j in range(8):
        for i, v in enumerate(vs):
            b = b.at[DE * j:DE * (j + 1), 8 * i + j].set(v)
    rb = (E // 8) // 40
    outs = pl.pallas_call(
        _elog_body,
        grid=(40,),
        in_specs=[
            pl.BlockSpec((rb, 8 * DE), lambda i: (i, 0)),
            pl.BlockSpec((8 * DE, 24), lambda i: (0, 0)),
        ],
        out_specs=[pl.BlockSpec((rb, 8), lambda i: (i, 0))] * 3,
        out_shape=[jax.ShapeDtypeStruct((E // 8, 8), jnp.float32)] * 3,
    )(ear, b)
    return [o.reshape(E) for o in outs]


def _mlp_body(g_ref, ea_ref, ae_ref, m2_ref, q_ref):
    z = g_ref[...] + jnp.dot(ea_ref[...], ae_ref[...],
                             preferred_element_type=jnp.float32)
    q_ref[...] = jnp.dot(jnp.maximum(z, 0.0), m2_ref[...],
                         preferred_element_type=jnp.float32)


def _final_mlp(g, edge_attr, ae_w, m2w, m2b):
    rb = E // EBLK
    q = pl.pallas_call(
        _mlp_body,
        grid=(EBLK,),
        in_specs=[
            pl.BlockSpec((rb, H), lambda i: (i, 0)),
            pl.BlockSpec((rb, DE), lambda i: (i, 0)),
            pl.BlockSpec((DE, H), lambda i: (0, 0)),
            pl.BlockSpec((H, 1), lambda i: (0, 0)),
        ],
        out_specs=pl.BlockSpec((rb, 1), lambda i: (i, 0)),
        out_shape=jax.ShapeDtypeStruct((E, 1), jnp.float32),
    )(g, edge_attr, ae_w, m2w)
    return q[:, 0] + m2b[0]


def kernel(node_x, edge_index, edge_attr, batch, W0, We0, as0, ad0, ae0, b0,
           W1, We1, as1, ad1, ae1, b1, W2, We2, as2, ad2, ae2, b2,
           M1w, M1b, M2w, M2b):
    src, dst = edge_index[0], edge_index[1]

    el0, el1, el2 = _elogs(edge_attr, [We0 @ ae0, We1 @ ae1, We2 @ ae2])

    h0, s0, d0 = _proj(node_x, W0, as0, ad0)
    p0, den0 = _sc_layer(edge_index, el0, h0, s0, d0)
    h1, s1, d1 = _combine_proj(p0, den0.reshape(NC, NDP)[:, :N].T, b0, W1, as1, ad1)
    p1, den1 = _sc_layer(edge_index, el1, h1, s1, d1)
    h2, s2, d2 = _combine_proj(p1, den1.reshape(NC, NDP)[:, :N].T, b1, W2, as2, ad2)
    p2, den2 = _sc_layer(edge_index, el2, h2, s2, d2)

    a1_w = M1w[:EMB]
    a2_w = M1w[EMB:2 * EMB]
    ae_w = M1w[2 * EMB:2 * EMB + DE]
    ac_w = M1w[2 * EMB + DE:]

    emb, u1, u2 = _combine2(p2, den2.reshape(NC, NDP)[:, :N].T, b2, a1_w, a2_w)

    psum, pmax, pcnt = _sc_pool(emb, batch)

    u1p = _ctx_fold(u1, batch, psum, pmax, pcnt, ac_w, M1b)
    g = _sc_gather2(src, dst, u1p, u2)
    return _final_mlp(g, edge_attr, ae_w, M2w, M2b)


# speedup vs baseline: 1.0015x; 1.0015x over previous
import jax
import jax.numpy as jnp
from jax import lax
from jax.experimental import pallas as pl
from jax.experimental.pallas import tpu as pltpu
from jax.experimental.pallas import tpu_sc as plsc

N = 10000
E = 320000
D = 128
DE = 16
H = 128
EMB = 128
G = 64

NC = 2
NS = 16
NW = NC * NS

CK = 128
CKG = 512
NCHUNKG = E // CKG
RFULLG = NCHUNKG // NW
RREMG = NCHUNKG - RFULLG * NW
NCHUNK = E // CK
RFULL = NCHUNK // NW
RREM = NCHUNK - RFULL * NW

NPW = 320
NP = NPW * NW
GP = 72

NDP = 10240
NTILE = 624
NREST = N - NTILE * NS
_HOPS = [(k, CK) for k in range(NTILE // CK)]
if NTILE % CK:
    _HOPS.append((NTILE // CK, NTILE % CK))

NBLK = 10
EBLK = 64


def _mesh():
    return plsc.VectorSubcoreMesh(core_axis_name="c", subcore_axis_name="s")


CKB = 1280
NCHUNKB = E // CKB
RFULLB = NCHUNKB // NW
RREMB = NCHUNKB - RFULLB * NW


def _sc_phaseB_body(ei_hbm, elog_hbm, s_hbm, d_hbm, exv_hbm, denp_hbm,
                    s_v, d_v, den_v, sdb, elb, exb, dstage, densum, den_sh):
    c = lax.axis_index("c")
    s = lax.axis_index("s")
    wid = s * NC + c

    pltpu.sync_copy(s_hbm, s_v)
    pltpu.sync_copy(d_hbm, d_v)

    zero16 = jnp.zeros((16,), jnp.float32)

    def _zero_den(g, carry):
        den_v[pl.ds(g * 16, 16)] = zero16
        return carry
    lax.fori_loop(0, NDP // 16, _zero_den, 0)

    def _chunk(chunk_id):
        base = chunk_id * CKB
        pltpu.sync_copy(ei_hbm.at[:, pl.ds(base, CKB)], sdb)
        pltpu.sync_copy(elog_hbm.at[pl.ds(base, CKB)], elb)

        def _grp(g, carry):
            off = g * 16
            src16 = sdb[0, pl.ds(off, 16)]
            dst16 = sdb[1, pl.ds(off, 16)]
            l16 = (plsc.load_gather(s_v, [src16])
                   + plsc.load_gather(d_v, [dst16])
                   + elb[pl.ds(off, 16)])
            l16 = jnp.where(l16 >= 0.0, l16, l16 * 0.2)
            ex16 = jnp.exp(l16)
            exb[pl.ds(off, 16)] = ex16
            plsc.addupdate_scatter(den_v, [dst16], ex16)
            return carry
        lax.fori_loop(0, CKB // 16, _grp, 0)
        pltpu.sync_copy(exb, exv_hbm.at[pl.ds(base, CKB)])

    def _round(r, carry):
        _chunk(r * NW + wid)
        return carry
    lax.fori_loop(0, RFULLB, _round, 0)

    @pl.when(wid < RREMB)
    def _tail():
        _chunk(RFULLB * NW + wid)

    pltpu.sync_copy(den_v, den_sh.at[s])
    plsc.subcore_barrier()
    r0 = s * 640
    pltpu.sync_copy(den_sh.at[:, pl.ds(r0, 640)], dstage)

    def _redgrp(g, carry):
        acc = dstage[0, pl.ds(g * 16, 16)]
        for tt in range(1, NS):
            acc = acc + dstage[tt, pl.ds(g * 16, 16)]
        densum[pl.ds(g * 16, 16)] = acc
        return carry
    lax.fori_loop(0, 40, _redgrp, 0)
    pltpu.sync_copy(densum, denp_hbm.at[c, 0, pl.ds(r0, 640)])


def _sc_phaseB(ei, elog, s, d):
    k = pl.kernel(
        _sc_phaseB_body,
        out_type=(
            jax.ShapeDtypeStruct((E,), jnp.float32),
            jax.ShapeDtypeStruct((NC, 1, NDP), jnp.float32),
        ),
        mesh=_mesh(),
        compiler_params=pltpu.CompilerParams(needs_layout_passes=False),
        scratch_types=[
            pltpu.VMEM((N,), jnp.float32),
            pltpu.VMEM((N,), jnp.float32),
            pltpu.VMEM((NDP,), jnp.float32),
            pltpu.VMEM((2, CKB), jnp.int32),
            pltpu.VMEM((CKB,), jnp.float32),
            pltpu.VMEM((CKB,), jnp.float32),
            pltpu.VMEM((NS, 640), jnp.float32),
            pltpu.VMEM((640,), jnp.float32),
            pltpu.VMEM_SHARED((NS, NDP), jnp.float32),
        ],
    )
    return k(ei, elog, s, d)


_NCH = NCHUNK
_RF = _NCH // NW
_RREMC = _NCH - _RF * NW


def _sc_phaseC_body(ei_hbm, exv_hbm, h_hbm, outp_hbm,
                    sdb2, dsc2, exb2, rows2, out_sh,
                    sem_g0, sem_g1, sem_s0, sem_s1, sem_i0, sem_i1):
    c = lax.axis_index("c")
    s = lax.axis_index("s")
    wid = s * NC + c
    sem_g = (sem_g0, sem_g1)
    sem_s = (sem_s0, sem_s1)
    sem_i = (sem_i0, sem_i1)

    zero16 = jnp.zeros((16,), jnp.float32)

    def _zero_rows(r, carry):
        for j in range(8):
            rows2[0, r, pl.ds(j * 16, 16)] = zero16
        return carry
    lax.fori_loop(0, CK, _zero_rows, 0)

    for k, nr in _HOPS:
        pltpu.sync_copy(rows2.at[0, pl.ds(0, nr)],
                        out_sh.at[pl.ds(s * NTILE + k * CK, nr)])

    @pl.when(s == 0)
    def _zero_rest():
        pltpu.sync_copy(rows2.at[0, pl.ds(0, NREST)],
                        out_sh.at[pl.ds(NS * NTILE, NREST)])
    plsc.subcore_barrier()

    def _cid(k):
        return k * NW + wid

    def _fire_stages(k, b):
        base = _cid(k) * CK
        pltpu.async_copy(ei_hbm.at[:, pl.ds(base, CK)], sdb2.at[b], sem_i[b])
        pltpu.async_copy(exv_hbm.at[pl.ds(base, CK)],
                         exb2.at[b, pl.ds(0, CK)], sem_i[b])

    def _drain_stages(b):
        pltpu.make_async_copy(ei_hbm.at[:, pl.ds(0, CK)], sdb2.at[b],
                              sem_i[b]).wait()
        pltpu.make_async_copy(exv_hbm.at[pl.ds(0, CK)],
                              exb2.at[b, pl.ds(0, CK)], sem_i[b]).wait()

    def _fire_gather(k, b):
        pltpu.async_copy(h_hbm.at[sdb2.at[b, 0]], rows2.at[b], sem_g[b])

    def _drain_gather(b):
        pltpu.make_async_copy(h_hbm.at[pl.ds(0, CK)], rows2.at[b],
                              sem_g[b]).wait()

    def _fire_scatter(b):
        pltpu.async_copy(rows2.at[b], out_sh.at[dsc2.at[b, 0]], sem_s[b],
                         add=True)

    def _drain_scatter(b):
        pltpu.make_async_copy(h_hbm.at[pl.ds(0, CK)], rows2.at[b],
                              sem_s[b]).wait()

    def _scale(b):
        def _one(e, carry):
            a16 = jnp.broadcast_to(exb2[b, pl.ds(e, 16)][0], (16,))
            for j in range(8):
                v = rows2[b, e, pl.ds(j * 16, 16)]
                rows2[b, e, pl.ds(j * 16, 16)] = v * a16
            return carry
        lax.fori_loop(0, CK, _one, 0, unroll=2)

    base0 = _cid(0) * CK
    pltpu.sync_copy(ei_hbm.at[:, pl.ds(base0, CK)], sdb2.at[0])
    pltpu.sync_copy(exv_hbm.at[pl.ds(base0, CK)], exb2.at[0, pl.ds(0, CK)])
    _fire_gather(0, 0)
    _fire_stages(1, 1)

    def _pair(r, carry):
        for b in (0, 1):
            k = r * 2 + b
            _drain_gather(b)
            for g in range(CK // 16):
                dsc2[b, 0, pl.ds(g * 16, 16)] = sdb2[b, 1, pl.ds(g * 16, 16)]
            _scale(b)
            _fire_scatter(b)

            @pl.when(k + 2 < _RF)
            def _pf_stages():
                _fire_stages(k + 2, b)

            @pl.when(k + 1 < _RF)
            def _pf_gather():
                @pl.when(k >= 1)
                def _w_sct():
                    _drain_scatter(1 - b)
                _drain_stages(1 - b)
                _fire_gather(k + 1, 1 - b)
        return carry
    lax.fori_loop(0, _RF // 2, _pair, 0)

    _drain_scatter(0)
    _drain_scatter(1)

    @pl.when(wid < _RREMC)
    def _tail():
        base = (_RF * NW + wid) * CK
        pltpu.sync_copy(ei_hbm.at[:, pl.ds(base, CK)], sdb2.at[0])
        pltpu.sync_copy(exv_hbm.at[pl.ds(base, CK)], exb2.at[0, pl.ds(0, CK)])
        pltpu.sync_copy(h_hbm.at[sdb2.at[0, 0]], rows2.at[0])
        for g in range(CK // 16):
            dsc2[0, 0, pl.ds(g * 16, 16)] = sdb2[0, 1, pl.ds(g * 16, 16)]
        _scale(0)
        pltpu.sync_copy(rows2.at[0], out_sh.at[dsc2.at[0, 0]], add=True)

    plsc.subcore_barrier()
    for k, nr in _HOPS:
        r0 = s * NTILE + k * CK
        pltpu.sync_copy(out_sh.at[pl.ds(r0, nr)], rows2.at[0, pl.ds(0, nr)])
        pltpu.sync_copy(rows2.at[0, pl.ds(0, nr)],
                        outp_hbm.at[c, pl.ds(r0, nr)])

    @pl.when(s == 0)
    def _drain_rest():
        pltpu.sync_copy(out_sh.at[pl.ds(NS * NTILE, NREST)],
                        rows2.at[0, pl.ds(0, NREST)])
        pltpu.sync_copy(rows2.at[0, pl.ds(0, NREST)],
                        outp_hbm.at[c, pl.ds(NS * NTILE, NREST)])


def _sc_phaseC(ei, exv, h):
    k = pl.kernel(
        _sc_phaseC_body,
        out_type=jax.ShapeDtypeStruct((NC, N, H), jnp.float32),
        mesh=_mesh(),
        compiler_params=pltpu.CompilerParams(needs_layout_passes=False),
        scratch_types=[
            pltpu.VMEM((2, 2, CK), jnp.int32),
            pltpu.VMEM((2, 1, CK), jnp.int32),
            pltpu.VMEM((2, CK + 16), jnp.float32),
            pltpu.VMEM((2, CK, H), jnp.float32),
            pltpu.VMEM_SHARED((N, H), jnp.float32),
            pltpu.SemaphoreType.DMA,
            pltpu.SemaphoreType.DMA,
            pltpu.SemaphoreType.DMA,
            pltpu.SemaphoreType.DMA,
            pltpu.SemaphoreType.DMA,
            pltpu.SemaphoreType.DMA,
        ],
    )
    return k(ei, exv, h)


def _sc_layer(ei, elog, h, s, d):
    exv, denp = _sc_phaseB(ei, elog, s, d)
    outp = _sc_phaseC(ei, exv, h)
    return outp, denp


def _sc_pool_body(emb_hbm, batch_hbm, psum_hbm, pmax_hbm, pcnt_hbm,
                  ebuf, bbuf, tsum, tmax, tcnt):
    c = lax.axis_index("c")
    s = lax.axis_index("s")
    wid = s * NC + c

    base = jnp.minimum(wid * NPW, N - NPW)
    skip = wid * NPW - base
    pltpu.sync_copy(emb_hbm.at[pl.ds(base, NPW)], ebuf)
    pltpu.sync_copy(batch_hbm.at[pl.ds(base, NPW)], bbuf.at[pl.ds(0, NPW)])

    zero16 = jnp.zeros((16,), jnp.float32)
    neg16 = jnp.full((16,), -3.0e38, jnp.float32)

    def _init(g, carry):
        for j in range(8):
            tsum[g, pl.ds(j * 16, 16)] = zero16
            tmax[g, pl.ds(j * 16, 16)] = neg16
        return carry
    lax.fori_loop(0, GP, _init, 0)

    def _zero_cnt(g, carry):
        tcnt[pl.ds(g * 16, 16)] = zero16
        return carry
    lax.fori_loop(0, 6, _zero_cnt, 0)

    def _node(i, carry):
        lane0 = (lax.iota(jnp.int32, 16) == 0).astype(jnp.float32)
        b = bbuf[pl.ds(i, 16)][0]
        for j in range(8):
            row = ebuf[i, pl.ds(j * 16, 16)]
            ts = tsum[b, pl.ds(j * 16, 16)]
            tsum[b, pl.ds(j * 16, 16)] = ts + row
            tm = tmax[b, pl.ds(j * 16, 16)]
            tmax[b, pl.ds(j * 16, 16)] = jnp.maximum(tm, row)
        cn = tcnt[pl.ds(b, 16)]
        tcnt[pl.ds(b, 16)] = cn + lane0
        return carry
    lax.fori_loop(skip, NPW, _node, 0)

    pltpu.sync_copy(tsum, psum_hbm.at[wid])
    pltpu.sync_copy(tmax, pmax_hbm.at[wid])
    pltpu.sync_copy(tcnt.at[pl.ds(0, GP)], pcnt_hbm.at[wid, 0])


def _sc_pool(emb_p, batch_p):
    k = pl.kernel(
        _sc_pool_body,
        out_type=(
            jax.ShapeDtypeStruct((NW, GP, EMB), jnp.float32),
            jax.ShapeDtypeStruct((NW, GP, EMB), jnp.float32),
            jax.ShapeDtypeStruct((NW, 1, GP), jnp.float32),
        ),
        mesh=_mesh(),
        compiler_params=pltpu.CompilerParams(needs_layout_passes=False),
        scratch_types=[
            pltpu.VMEM((NPW, EMB), jnp.float32),
            pltpu.VMEM((NPW + 16,), jnp.int32),
            pltpu.VMEM((GP, EMB), jnp.float32),
            pltpu.VMEM((GP, EMB), jnp.float32),
            pltpu.VMEM((96,), jnp.float32),
        ],
    )
    return k(emb_p, batch_p)


def _sc_gather2_body(src_hbm, dst_hbm, u1_hbm, u2_hbm, g_hbm,
                     srcb, dstb, rows, sem):
    c = lax.axis_index("c")
    s = lax.axis_index("s")
    wid = s * NC + c

    def _chunk(chunk_id):
        base = chunk_id * CKG
        pltpu.sync_copy(src_hbm.at[pl.ds(base, CKG)], srcb.at[0])
        pltpu.sync_copy(dst_hbm.at[pl.ds(base, CKG)], dstb.at[0])
        pltpu.sync_copy(u1_hbm.at[srcb.at[0]], rows)
        pltpu.async_copy(u2_hbm.at[dstb.at[0]], rows, sem, add=True).wait()
        pltpu.sync_copy(rows, g_hbm.at[pl.ds(base, CKG)])

    def _round(r, carry):
        _chunk(r * NW + wid)
        return carry
    lax.fori_loop(0, RFULLG, _round, 0)

    @pl.when(wid < RREMG)
    def _tail():
        _chunk(RFULLG * NW + wid)


def _sc_gather2(src, dst, u1p, u2):
    k = pl.kernel(
        _sc_gather2_body,
        out_type=jax.ShapeDtypeStruct((E, H), jnp.float32),
        mesh=_mesh(),
        compiler_params=pltpu.CompilerParams(needs_layout_passes=False),
        scratch_types=[
            pltpu.VMEM((1, CKG), jnp.int32),
            pltpu.VMEM((1, CKG), jnp.int32),
            pltpu.VMEM((CKG, H), jnp.float32),
            pltpu.SemaphoreType.DMA,
        ],
    )
    return k(src, dst, u1p, u2)


def _proj_body(act_ref, w_ref, a2_ref, h_ref, sd_ref):
    h = jnp.dot(act_ref[...], w_ref[...], preferred_element_type=jnp.float32)
    h_ref[...] = h
    sd_ref[...] = jnp.dot(h, a2_ref[...], preferred_element_type=jnp.float32)


def _proj(act, w, a_s, a_d):
    a2 = jnp.stack([a_s, a_d], axis=1)
    rb = N // NBLK
    h, sd = pl.pallas_call(
        _proj_body,
        grid=(NBLK,),
        in_specs=[
            pl.BlockSpec((rb, D), lambda i: (i, 0)),
            pl.BlockSpec((D, H), lambda i: (0, 0)),
            pl.BlockSpec((H, 2), lambda i: (0, 0)),
        ],
        out_specs=[
            pl.BlockSpec((rb, H), lambda i: (i, 0)),
            pl.BlockSpec((rb, 2), lambda i: (i, 0)),
        ],
        out_shape=[
            jax.ShapeDtypeStruct((N, H), jnp.float32),
            jax.ShapeDtypeStruct((N, 2), jnp.float32),
        ],
    )(act, w, a2)
    return h, sd[:, 0], sd[:, 1]


def _combine_proj_body(p_ref, dent_ref, b_ref, w_ref, a2_ref, h_ref, sd_ref):
    dr = 1.0 / (jnp.sum(dent_ref[...], axis=1) + 1e-16)
    o = (p_ref[0] + p_ref[1]) * dr[:, None] + b_ref[...][None, :]
    act = jnp.maximum(o, 0.0)
    h = jnp.dot(act, w_ref[...], preferred_element_type=jnp.float32)
    h_ref[...] = h
    sd_ref[...] = jnp.dot(h, a2_ref[...], preferred_element_type=jnp.float32)


def _combine_proj(outp, dent, b, w, a_s, a_d):
    a2 = jnp.stack([a_s, a_d], axis=1)
    rb = N // NBLK
    h, sd = pl.pallas_call(
        _combine_proj_body,
        grid=(NBLK,),
        in_specs=[
            pl.BlockSpec((NC, rb, H), lambda i: (0, i, 0)),
            pl.BlockSpec((rb, NC), lambda i: (i, 0)),
            pl.BlockSpec((H,), lambda i: (0,)),
            pl.BlockSpec((D, H), lambda i: (0, 0)),
            pl.BlockSpec((H, 2), lambda i: (0, 0)),
        ],
        out_specs=[
            pl.BlockSpec((rb, H), lambda i: (i, 0)),
            pl.BlockSpec((rb, 2), lambda i: (i, 0)),
        ],
        out_shape=[
            jax.ShapeDtypeStruct((N, H), jnp.float32),
            jax.ShapeDtypeStruct((N, 2), jnp.float32),
        ],
    )(outp, dent, b, w, a2)
    return h, sd[:, 0], sd[:, 1]


def _combine2_body(p_ref, dent_ref, b_ref, aw_ref, emb_ref, u_ref):
    dr = 1.0 / (jnp.sum(dent_ref[...], axis=1) + 1e-16)
    emb = (p_ref[0] + p_ref[1]) * dr[:, None] + b_ref[...][None, :]
    emb_ref[...] = emb
    u_ref[...] = jnp.dot(emb, aw_ref[...], preferred_element_type=jnp.float32)


def _combine2(outp, dent, b, a1w, a2w):
    aw = jnp.concatenate([a1w, a2w], axis=1)
    rb = N // NBLK
    emb, u = pl.pallas_call(
        _combine2_body,
        grid=(NBLK,),
        in_specs=[
            pl.BlockSpec((NC, rb, H), lambda i: (0, i, 0)),
            pl.BlockSpec((rb, NC), lambda i: (i, 0)),
            pl.BlockSpec((H,), lambda i: (0,)),
            pl.BlockSpec((H, 2 * H), lambda i: (0, 0)),
        ],
        out_specs=[
            pl.BlockSpec((rb, H), lambda i: (i, 0)),
            pl.BlockSpec((rb, 2 * H), lambda i: (i, 0)),
        ],
        out_shape=[
            jax.ShapeDtypeStruct((N, H), jnp.float32),
            jax.ShapeDtypeStruct((N, 2 * H), jnp.float32),
        ],
    )(outp, dent, b, aw)
    return emb, u[:, :H], u[:, H:]


def _ctx_fold_body(u1_ref, b2d_ref, ps_ref, pm_ref, pc_ref, ac_ref,
                   m1b_ref, o_ref):
    cnt = jnp.sum(pc_ref[...], axis=(0, 1))[:G]
    gsum = jnp.sum(ps_ref[...], axis=0)[:G]
    gmax = jnp.max(pm_ref[...], axis=0)[:G]
    gmean = gsum / jnp.clip(cnt, 1.0)[:, None]
    gmax = jnp.where(cnt[:, None] > 0.5, gmax, 0.0)
    ctx = jnp.concatenate([gmean, gmax], axis=1)
    cb = jnp.dot(ctx, ac_ref[...], preferred_element_type=jnp.float32) \
        + m1b_ref[...][None, :]
    oh = (b2d_ref[...] == lax.broadcasted_iota(jnp.int32, (1, G), 1))
    o_ref[...] = u1_ref[...] + jnp.dot(
        oh.astype(jnp.float32), cb, preferred_element_type=jnp.float32)


def _ctx_fold(u1, batch, psum, pmax, pcnt, ac_w, m1b):
    rb = N // NBLK
    return pl.pallas_call(
        _ctx_fold_body,
        grid=(NBLK,),
        in_specs=[
            pl.BlockSpec((rb, H), lambda i: (i, 0)),
            pl.BlockSpec((rb, 1), lambda i: (i, 0)),
            pl.BlockSpec((NW, GP, EMB), lambda i: (0, 0, 0)),
            pl.BlockSpec((NW, GP, EMB), lambda i: (0, 0, 0)),
            pl.BlockSpec((NW, 1, GP), lambda i: (0, 0, 0)),
            pl.BlockSpec((2 * EMB, H), lambda i: (0, 0)),
            pl.BlockSpec((H,), lambda i: (0,)),
        ],
        out_specs=pl.BlockSpec((rb, H), lambda i: (i, 0)),
        out_shape=jax.ShapeDtypeStruct((N, H), jnp.float32),
    )(u1, batch[:, None], psum, pmax, pcnt, ac_w, m1b)


def _elog_body(ea_ref, b_ref, o0_ref, o1_ref, o2_ref):
    o = jnp.dot(ea_ref[...], b_ref[...], preferred_element_type=jnp.float32)
    o0_ref[...] = o[:, 0:8]
    o1_ref[...] = o[:, 8:16]
    o2_ref[...] = o[:, 16:24]


def _elogs(edge_attr, vs):
    ear = edge_attr.reshape(E // 8, 8 * DE)
    b = jnp.zeros((8 * DE, 24), jnp.float32)
    for j in range(8):
        for i, v in enumerate(vs):
            b = b.at[DE * j:DE * (j + 1), 8 * i + j].set(v)
    rb = (E // 8) // 40
    outs = pl.pallas_call(
        _elog_body,
        grid=(40,),
        in_specs=[
            pl.BlockSpec((rb, 8 * DE), lambda i: (i, 0)),
            pl.BlockSpec((8 * DE, 24), lambda i: (0, 0)),
        ],
        out_specs=[pl.BlockSpec((rb, 8), lambda i: (i, 0))] * 3,
        out_shape=[jax.ShapeDtypeStruct((E // 8, 8), jnp.float32)] * 3,
    )(ear, b)
    return [o.reshape(E) for o in outs]


def _mlp_body(g_ref, ea_ref, ae_ref, m2_ref, q_ref):
    z = g_ref[...] + jnp.dot(ea_ref[...], ae_ref[...],
                             preferred_element_type=jnp.float32)
    q_ref[...] = jnp.dot(jnp.maximum(z, 0.0), m2_ref[...],
                         preferred_element_type=jnp.float32)


def _final_mlp(g, edge_attr, ae_w, m2w, m2b):
    rb = E // EBLK
    q = pl.pallas_call(
        _mlp_body,
        grid=(EBLK,),
        in_specs=[
            pl.BlockSpec((rb, H), lambda i: (i, 0)),
            pl.BlockSpec((rb, DE), lambda i: (i, 0)),
            pl.BlockSpec((DE, H), lambda i: (0, 0)),
            pl.BlockSpec((H, 1), lambda i: (0, 0)),
        ],
        out_specs=pl.BlockSpec((rb, 1), lambda i: (i, 0)),
        out_shape=jax.ShapeDtypeStruct((E, 1), jnp.float32),
    )(g, edge_attr, ae_w, m2w)
    return q[:, 0] + m2b[0]


def kernel(node_x, edge_index, edge_attr, batch, W0, We0, as0, ad0, ae0, b0,
           W1, We1, as1, ad1, ae1, b1, W2, We2, as2, ad2, ae2, b2,
           M1w, M1b, M2w, M2b):
    src, dst = edge_index[0], edge_index[1]

    el0, el1, el2 = _elogs(edge_attr, [We0 @ ae0, We1 @ ae1, We2 @ ae2])

    h0, s0, d0 = _proj(node_x, W0, as0, ad0)
    p0, den0 = _sc_layer(edge_index, el0, h0, s0, d0)
    h1, s1, d1 = _combine_proj(p0, den0.reshape(NC, NDP)[:, :N].T, b0, W1, as1, ad1)
    p1, den1 = _sc_layer(edge_index, el1, h1, s1, d1)
    h2, s2, d2 = _combine_proj(p1, den1.reshape(NC, NDP)[:, :N].T, b1, W2, as2, ad2)
    p2, den2 = _sc_layer(edge_index, el2, h2, s2, d2)

    a1_w = M1w[:EMB]
    a2_w = M1w[EMB:2 * EMB]
    ae_w = M1w[2 * EMB:2 * EMB + DE]
    ac_w = M1w[2 * EMB + DE:]

    emb, u1, u2 = _combine2(p2, den2.reshape(NC, NDP)[:, :N].T, b2, a1_w, a2_w)

    psum, pmax, pcnt = _sc_pool(emb, batch)

    u1p = _ctx_fold(u1, batch, psum, pmax, pcnt, ac_w, M1b)
    g = _sc_gather2(src, dst, u1p, u2)
    return _final_mlp(g, edge_attr, ae_w, M2w, M2b)
